# baseline (speedup 1.0000x reference)
import functools

import jax
import jax.numpy as jnp
from jax import lax
from jax.experimental import pallas as pl
from jax.experimental.pallas import tpu as pltpu
from jax.experimental.pallas import tpu_sc as plsc

_NT_M = 119
_NP_M = 16
_EMB = 128
_IN = _NT_M + _NP_M
_BLOCK = 10000
_N = 100000

_NW = 32
_SB = 200
_NBLK = _N // _SB
_G = _SB // 8


def _tc_body(x_ref, wpad_ref, b_ref, idx_ref, hnp_ref):
    x = x_ref[...]
    col = jax.lax.broadcasted_iota(jnp.int32, (1, _IN), 1)
    arange_nt = jnp.where(col < _NT_M, col, 0).astype(jnp.float32)
    idx_f = jnp.sum(x * arange_nt, axis=1)
    idx = jnp.clip(idx_f.astype(jnp.int32), 0, _NT_M - 1)
    idx_ref[...] = idx.reshape(1, 1, -1)
    hnp_ref[...] = jnp.dot(x, wpad_ref[...], preferred_element_type=jnp.float32) + b_ref[...]


def _tc_stage(X, wpad, b2):
    n = X.shape[0]
    grid = n // _BLOCK
    return pl.pallas_call(
        _tc_body,
        grid=(grid,),
        in_specs=[
            pl.BlockSpec((_BLOCK, _IN), lambda i: (i, 0)),
            pl.BlockSpec((_IN, _EMB), lambda i: (0, 0)),
            pl.BlockSpec((1, _EMB), lambda i: (0, 0)),
        ],
        out_specs=[
            pl.BlockSpec((1, 1, _BLOCK), lambda i: (i, 0, 0)),
            pl.BlockSpec((_BLOCK, _EMB), lambda i: (i, 0)),
        ],
        out_shape=[
            jax.ShapeDtypeStruct((grid, 1, _BLOCK), jnp.int32),
            jax.ShapeDtypeStruct((n, _EMB), jnp.float32),
        ],
    )(X, wpad, b2)


_PSB = 400
_PNBLK = _N // _PSB


@functools.partial(
    pl.kernel,
    mesh=plsc.VectorSubcoreMesh(core_axis_name="c", subcore_axis_name="s"),
    out_type=jax.ShapeDtypeStruct((_N, _IN), jnp.float32),
    scratch_types=[
        pltpu.VMEM((_PSB, _IN), jnp.float32),
        pltpu.SemaphoreType.DMA,
    ],
)
def _sc_probe(x_hbm, out_hbm, buf_v, sem):
    wid = lax.axis_index("s") * 2 + lax.axis_index("c")

    def body(j, carry):
        blk = j * _NW + wid

        @pl.when(blk < _PNBLK)
        def _():
            row0 = blk * _PSB
            pltpu.sync_copy(x_hbm.at[pl.ds(row0, _PSB), :], buf_v)
            pltpu.sync_copy(buf_v, out_hbm.at[pl.ds(row0, _PSB), :])

        return carry

    lax.fori_loop(0, (_PNBLK + _NW - 1) // _NW, body, 0)


def kernel(X, nt_emb, W, b):
    wpad = jnp.zeros((_IN, _EMB), jnp.float32).at[_NT_M:, :].set(W.T)
    b2 = b.reshape(1, _EMB)
    idx3, hnp = _tc_stage(X, wpad, b2)
    return _sc_probe(X), idx3, hnp


# speedup vs baseline: 2.0397x; 2.0397x over previous
import jax
import jax.numpy as jnp
from jax.experimental import pallas as pl

_NT_M = 119
_NP_M = 16
_EMB = 128
_IN = _NT_M + _NP_M
_BLOCK = 10000


def _body(x_ref, table_ref, wpad_ref, b_ref, out_ref):
    x = x_ref[...]
    col = jax.lax.broadcasted_iota(jnp.int32, (1, _IN), 1)
    arange_nt = jnp.where(col < _NT_M, col, 0).astype(jnp.float32)
    idx_f = jnp.sum(x * arange_nt, axis=1, keepdims=True)
    idx = jnp.clip(idx_f.astype(jnp.int32), 0, _NT_M - 1)
    cols = jax.lax.broadcasted_iota(jnp.int32, (x.shape[0], _EMB), 1)
    onehot = (cols == idx).astype(jnp.float32)
    h_nt = jnp.dot(onehot, table_ref[...], preferred_element_type=jnp.float32)
    h_np = jnp.dot(x, wpad_ref[...], preferred_element_type=jnp.float32) + b_ref[...]
    out_ref[...] = jnp.concatenate([h_nt, h_np], axis=1)


def kernel(X, nt_emb, W, b):
    n = X.shape[0]
    table = jnp.zeros((_EMB, _EMB), jnp.float32).at[:_NT_M, :].set(nt_emb)
    wpad = jnp.zeros((_IN, _EMB), jnp.float32).at[_NT_M:, :].set(W.T)
    b2 = b.reshape(1, _EMB)
    grid = (n + _BLOCK - 1) // _BLOCK
    return pl.pallas_call(
        _body,
        grid=(grid,),
        in_specs=[
            pl.BlockSpec((_BLOCK, _IN), lambda i: (i, 0)),
            pl.BlockSpec((_EMB, _EMB), lambda i: (0, 0)),
            pl.BlockSpec((_IN, _EMB), lambda i: (0, 0)),
            pl.BlockSpec((1, _EMB), lambda i: (0, 0)),
        ],
        out_specs=pl.BlockSpec((_BLOCK, 2 * _EMB), lambda i: (i, 0)),
        out_shape=jax.ShapeDtypeStruct((n, 2 * _EMB), jnp.float32),
    )(X, table, wpad, b2)
